# baseline (speedup 1.0000x reference)
import functools

import numpy as np
import jax
import jax.numpy as jnp
from jax import lax
from jax.experimental import pallas as pl
from jax.experimental.pallas import tpu as pltpu
from jax.experimental.pallas import tpu_sc as plsc

_F = 26
_D = 16
_B = 16384
_N = _B * _F
_NUM_EMB = 40000 * _F
_OFFS = np.arange(_F, dtype=np.int32) * 40000

_NC = 2
_NS = 16
_NW = _NC * _NS
_PER_W = _N // _NW
_REP = 208
_CHUNK = 1664
_NCH = _PER_W // _CHUNK

_OFFS_TILED = np.tile(_OFFS, _REP // _F)

_mesh = plsc.VectorSubcoreMesh(core_axis_name="c", subcore_axis_name="s")


@functools.partial(
    pl.kernel,
    mesh=_mesh,
    out_type=jax.ShapeDtypeStruct((_N, _D), jnp.float32),
    scratch_types=[
        pltpu.VMEM((_REP,), jnp.int32),
        pltpu.VMEM((_CHUNK,), jnp.int32),
        pltpu.VMEM((_CHUNK,), jnp.int32),
        pltpu.VMEM((_CHUNK, _D), jnp.float32),
        pltpu.SemaphoreType.DMA,
    ],
    compiler_params=pltpu.CompilerParams(use_tc_tiling_on_sc=False),
)
def _emb_lookup(weight_hbm, xflat_hbm, offs_hbm, out_hbm,
                off_v, x_v, idx_v, rows_v, sem):
    wid = lax.axis_index("s") * _NC + lax.axis_index("c")
    pltpu.sync_copy(offs_hbm, off_v)
    wbase = wid * _PER_W

    def chunk_body(c, carry):
        base = wbase + c * _CHUNK
        pltpu.sync_copy(xflat_hbm.at[pl.ds(base, _CHUNK)], x_v)
        for j in range(_CHUNK // 16):
            s = j * 16
            idx_v[pl.ds(s, 16)] = x_v[pl.ds(s, 16)] + off_v[pl.ds(s % _REP, 16)]
        pltpu.async_copy(weight_hbm.at[idx_v], rows_v, sem).wait()
        pltpu.sync_copy(rows_v, out_hbm.at[pl.ds(base, _CHUNK)])
        return carry

    lax.fori_loop(0, _NCH, chunk_body, 0)


def kernel(x, weight):
    xflat = x.reshape(-1)
    offs = jnp.asarray(_OFFS_TILED)
    out = _emb_lookup(weight, xflat, offs)
    return out.reshape(_B, _F, _D)


# speedup vs baseline: 2.7927x; 2.7927x over previous
import functools

import jax
import jax.numpy as jnp
from jax import lax
from jax.experimental import pallas as pl
from jax.experimental.pallas import tpu as pltpu
from jax.experimental.pallas import tpu_sc as plsc

_F = 26
_D = 16
_B = 16384
_V = 1040000
_C = 128
_NU = _F * (_B // _C)
_NW = 32
_UPW = _NU // _NW
_DPLANE = (_V // 128) * 1024

_mesh = plsc.VectorSubcoreMesh(core_axis_name="c", subcore_axis_name="s")


@functools.partial(
    pl.kernel,
    mesh=_mesh,
    out_type=jax.ShapeDtypeStruct((_F * _D * _B,), jnp.float32),
    scratch_types=[
        pltpu.VMEM((_C,), jnp.int32),
        pltpu.VMEM((8 * _C,), jnp.int32),
        pltpu.VMEM((8 * _C,), jnp.int32),
        pltpu.VMEM((8 * _C,), jnp.float32),
        pltpu.VMEM((8 * _C,), jnp.float32),
        pltpu.SemaphoreType.DMA,
        pltpu.SemaphoreType.DMA,
    ],
    compiler_params=pltpu.CompilerParams(use_tc_tiling_on_sc=True),
)
def _emb_lookup(w1d_hbm, xt_hbm, out_hbm,
                xv, idx0, idx1, dst0, dst1, sem0, sem1):
    wid = lax.axis_index("s") * 2 + lax.axis_index("c")

    def unit_body(i, carry):
        u = i * _NW + wid
        f = u // (_B // _C)
        b0 = (u % (_B // _C)) * _C
        pltpu.sync_copy(xt_hbm.at[f, pl.ds(b0, _C)], xv)
        off = f * 40000
        for g in range(_C // 16):
            r = xv[pl.ds(16 * g, 16)] + off
            base = ((r >> 7) << 10) + (r & 127)
            for s in range(8):
                idx0[pl.ds(s * _C + 16 * g, 16)] = base + (s * 128)
                idx1[pl.ds(s * _C + 16 * g, 16)] = base + (_DPLANE + s * 128)
        c0 = pltpu.async_copy(w1d_hbm.at[idx0], dst0, sem0)
        c1 = pltpu.async_copy(w1d_hbm.at[idx1], dst1, sem1)
        c0.wait()
        c1.wait()
        obase = f * (_D * _B) + (b0 // _C) * 1024
        pltpu.sync_copy(dst0, out_hbm.at[pl.ds(obase, 8 * _C)])
        pltpu.sync_copy(dst1, out_hbm.at[pl.ds(obase + 8 * _B, 8 * _C)])
        return carry

    lax.fori_loop(0, _UPW, unit_body, 0)


def kernel(x, weight):
    wt = weight.T
    w1d = (wt.reshape(2, 8, _V // 128, 128)
             .transpose(0, 2, 1, 3)
             .reshape(-1))
    xt = x.T
    out1d = _emb_lookup(w1d, xt)
    return (out1d.reshape(_F, 2, _B // _C, 8, _C)
                 .transpose(2, 4, 0, 1, 3)
                 .reshape(_B, _F, _D))


# speedup vs baseline: 3.5866x; 1.2843x over previous
import functools

import jax
import jax.numpy as jnp
from jax import lax
from jax.experimental import pallas as pl
from jax.experimental.pallas import tpu as pltpu
from jax.experimental.pallas import tpu_sc as plsc

_F = 26
_D = 16
_B = 16384
_V = 1040000
_C = 256
_RPF = _B // _C
_NU = _F * _RPF
_NW = 32
_UPW = _NU // _NW
_HC = 8 * _C
_DPLANE = (_V // 128) * 1024
_NBUF = 4

_mesh = plsc.VectorSubcoreMesh(core_axis_name="c", subcore_axis_name="s")

_scratch = []
for _ in range(_NBUF):
    _scratch += [
        pltpu.VMEM((_C,), jnp.int32),
        pltpu.VMEM((_HC,), jnp.int32),
        pltpu.VMEM((_HC,), jnp.int32),
        pltpu.VMEM((_HC,), jnp.float32),
        pltpu.VMEM((_HC,), jnp.float32),
        pltpu.SemaphoreType.DMA,
        pltpu.SemaphoreType.DMA,
        pltpu.SemaphoreType.DMA,
    ]


@functools.partial(
    pl.kernel,
    mesh=_mesh,
    out_type=jax.ShapeDtypeStruct((_F * _D * _B,), jnp.float32),
    scratch_types=_scratch,
    compiler_params=pltpu.CompilerParams(use_tc_tiling_on_sc=True),
)
def _emb_lookup(w1d_hbm, xt_hbm, out_hbm, *bufs):
    wid = lax.axis_index("s") * 2 + lax.axis_index("c")
    sets = [bufs[8 * i: 8 * i + 8] for i in range(_NBUF)]

    def stage(v, s, wait_writes):
        xv, idx0, idx1, dst0, dst1, g0, g1, ws = s
        u = v * _NW + wid
        f = u // _RPF
        b0 = (u % _RPF) * _C
        pltpu.sync_copy(xt_hbm.at[f, pl.ds(b0, _C)], xv)
        off = f * 40000
        if wait_writes:
            pltpu.make_async_copy(dst0, out_hbm.at[pl.ds(0, _HC)], ws).wait()
            pltpu.make_async_copy(dst1, out_hbm.at[pl.ds(0, _HC)], ws).wait()
        for g in range(_C // 16):
            r = xv[pl.ds(16 * g, 16)] + off
            base = ((r >> 7) << 10) + (r & 127)
            slot = (g // 8) * 1024 + (g % 8) * 16
            for sub in range(8):
                idx0[pl.ds(slot + sub * 128, 16)] = base + (sub * 128)
                idx1[pl.ds(slot + sub * 128, 16)] = base + (_DPLANE + sub * 128)
        pltpu.async_copy(w1d_hbm.at[idx0], dst0, g0)
        pltpu.async_copy(w1d_hbm.at[idx1], dst1, g1)

    def drain(v, s):
        xv, idx0, idx1, dst0, dst1, g0, g1, ws = s
        u = v * _NW + wid
        f = u // _RPF
        b0 = (u % _RPF) * _C
        obase = f * (_D * _B) + (b0 // 128) * 1024
        pltpu.make_async_copy(w1d_hbm.at[idx0], dst0, g0).wait()
        pltpu.make_async_copy(w1d_hbm.at[idx1], dst1, g1).wait()
        pltpu.async_copy(dst0, out_hbm.at[pl.ds(obase, _HC)], ws)
        pltpu.async_copy(dst1, out_hbm.at[pl.ds(obase + 8 * _B, _HC)], ws)

    stage(0, sets[0], False)
    stage(1, sets[1], False)
    stage(2, sets[2], False)
    drain(0, sets[0])
    stage(3, sets[3], False)
    for j in range(1, _NBUF):
        drain(j, sets[j])
        stage(j + 3, sets[(j + 3) % _NBUF], True)

    def steady(k, carry):
        v = k * _NBUF
        for j in range(_NBUF):
            drain(v + j, sets[j])
            stage(v + j + 3, sets[(j + 3) % _NBUF], True)
        return carry

    lax.fori_loop(1, _UPW // _NBUF - 1, steady, 0)

    v = _UPW - _NBUF
    drain(v, sets[v % _NBUF])
    stage(_UPW - 1, sets[(_UPW - 1) % _NBUF], True)
    for j in range(1, _NBUF):
        drain(v + j, sets[(v + j) % _NBUF])
    for s in sets:
        _, _, _, dst0, dst1, _, _, ws = s
        pltpu.make_async_copy(dst0, out_hbm.at[pl.ds(0, _HC)], ws).wait()
        pltpu.make_async_copy(dst1, out_hbm.at[pl.ds(0, _HC)], ws).wait()


def kernel(x, weight):
    wt = weight.T
    w1d = (wt.reshape(2, 8, _V // 128, 128)
             .transpose(0, 2, 1, 3)
             .reshape(-1))
    xt = x.T
    out1d = _emb_lookup(w1d, xt)
    return (out1d.reshape(_F, 2, _B // 128, 8, 128)
                 .transpose(2, 4, 0, 1, 3)
                 .reshape(_B, _F, _D))


# speedup vs baseline: 8.8267x; 2.4610x over previous
import functools

import jax
import jax.numpy as jnp
from jax import lax
from jax.experimental import pallas as pl
from jax.experimental.pallas import tpu as pltpu
from jax.experimental.pallas import tpu_sc as plsc

_F = 26
_D = 16
_B = 16384
_V = 1040000
_C = 256
_UPF = 1024 // _C
_HC = 8 * _C
_DPLANE = (_V // 128) * 1024
_NR = 314
_SPLANE = _NR * 1024
_SSLOT = 2 * _SPLANE
_SHARE = _SPLANE // 16
_FPC = _F // 2

_mesh = plsc.VectorSubcoreMesh(core_axis_name="c", subcore_axis_name="s")

_scratch = [pltpu.VMEM_SHARED((2 * _SSLOT,), jnp.float32),
            pltpu.SemaphoreType.DMA]
for _ in range(_UPF):
    _scratch += [
        pltpu.VMEM((_C,), jnp.int32),
        pltpu.VMEM((_HC,), jnp.int32),
        pltpu.VMEM((_HC,), jnp.int32),
        pltpu.VMEM((_HC,), jnp.float32),
        pltpu.VMEM((_HC,), jnp.float32),
        pltpu.SemaphoreType.DMA,
        pltpu.SemaphoreType.DMA,
        pltpu.SemaphoreType.DMA,
    ]


@functools.partial(
    pl.kernel,
    mesh=_mesh,
    out_type=jax.ShapeDtypeStruct((_F * _D * _B,), jnp.float32),
    scratch_types=_scratch,
    compiler_params=pltpu.CompilerParams(use_tc_tiling_on_sc=True),
)
def _emb_lookup(w1d_hbm, xt_hbm, out_hbm, spmem, asem, *bufs):
    core = lax.axis_index("c")
    tid = lax.axis_index("s")
    sets = [bufs[8 * i: 8 * i + 8] for i in range(_UPF)]

    def field_of(j):
        return 2 * j + core

    def rs_of(f):
        r0 = (40000 * f) >> 7
        return jnp.minimum(r0, (_V // 128) - _NR)

    def fire_phase_a(j, slot):
        f = field_of(j)
        rs = rs_of(f)
        for dpl in range(2):
            src = dpl * _DPLANE + rs * 1024 + tid * _SHARE
            dstw = slot * _SSLOT + dpl * _SPLANE + tid * _SHARE
            pltpu.async_copy(w1d_hbm.at[pl.ds(src, _SHARE)],
                             spmem.at[pl.ds(dstw, _SHARE)], asem)

    def wait_phase_a():
        for _ in range(2):
            pltpu.make_async_copy(w1d_hbm.at[pl.ds(0, _SHARE)],
                                  spmem.at[pl.ds(0, _SHARE)], asem).wait()

    def stage(u, s, f, addc):
        xv, idx0, idx1, dst0, dst1, g0, g1, ws = s
        b0 = tid * 1024 + u * _C
        pltpu.sync_copy(xt_hbm.at[f, pl.ds(b0, _C)], xv)
        off = 40000 * f
        for g in range(_C // 16):
            r = xv[pl.ds(16 * g, 16)] + off
            base = ((r >> 7) << 10) + (r & 127) + addc
            slot = (g // 8) * 1024 + (g % 8) * 16
            for sub in range(8):
                idx0[pl.ds(slot + sub * 128, 16)] = base + (sub * 128)
                idx1[pl.ds(slot + sub * 128, 16)] = base + (_SPLANE + sub * 128)
        pltpu.async_copy(spmem.at[idx0], dst0, g0)
        pltpu.async_copy(spmem.at[idx1], dst1, g1)

    def drain(u, s, f):
        xv, idx0, idx1, dst0, dst1, g0, g1, ws = s
        b0 = tid * 1024 + u * _C
        obase = f * (_D * _B) + (b0 // 128) * 1024
        pltpu.make_async_copy(spmem.at[idx0], dst0, g0).wait()
        pltpu.make_async_copy(spmem.at[idx1], dst1, g1).wait()
        pltpu.async_copy(dst0, out_hbm.at[pl.ds(obase, _HC)], ws)
        pltpu.async_copy(dst1, out_hbm.at[pl.ds(obase + 8 * _B, _HC)], ws)

    fire_phase_a(0, 0)

    def body(j, carry):
        slot = j & 1
        f = field_of(j)
        wait_phase_a()
        plsc.subcore_barrier()
        fire_phase_a(jnp.minimum(j + 1, _FPC - 1), 1 - slot)
        addc = slot * _SSLOT - rs_of(f) * 1024
        for u in range(_UPF):
            stage(u, sets[u], f, addc)
        for u in range(_UPF):
            drain(u, sets[u], f)
        for u in range(_UPF):
            _, _, _, dst0, dst1, _, _, ws = sets[u]
            pltpu.make_async_copy(dst0, out_hbm.at[pl.ds(0, _HC)], ws).wait()
            pltpu.make_async_copy(dst1, out_hbm.at[pl.ds(0, _HC)], ws).wait()
        return carry

    lax.fori_loop(0, _FPC, body, 0)
    wait_phase_a()


def kernel(x, weight):
    wt = weight.T
    w1d = (wt.reshape(2, 8, _V // 128, 128)
             .transpose(0, 2, 1, 3)
             .reshape(-1))
    xt = x.T
    out1d = _emb_lookup(w1d, xt)
    return (out1d.reshape(_F, 2, _B // 128, 8, 128)
                 .transpose(2, 4, 0, 1, 3)
                 .reshape(_B, _F, _D))


# speedup vs baseline: 9.5529x; 1.0823x over previous
import functools

import jax
import jax.numpy as jnp
from jax import lax
from jax.experimental import pallas as pl
from jax.experimental.pallas import tpu as pltpu
from jax.experimental.pallas import tpu_sc as plsc

_F = 26
_D = 16
_B = 16384
_V = 1040000
_C = 256
_UPF = 1024 // _C
_HC = 8 * _C
_DPLANE = (_V // 128) * 1024
_NR = 314
_SPLANE = _NR * 1024
_SSLOT = 2 * _SPLANE
_SHARE = _SPLANE // 16
_FPC = _F // 2

_mesh = plsc.VectorSubcoreMesh(core_axis_name="c", subcore_axis_name="s")

_scratch = [pltpu.VMEM_SHARED((2 * _SSLOT,), jnp.float32),
            pltpu.SemaphoreType.DMA,
            pltpu.VMEM((2 * 1024,), jnp.int32),
            pltpu.SemaphoreType.DMA]
for _ in range(_UPF):
    _scratch += [
        pltpu.VMEM((_HC,), jnp.int32),
        pltpu.VMEM((_HC,), jnp.int32),
        pltpu.VMEM((_HC,), jnp.float32),
        pltpu.VMEM((_HC,), jnp.float32),
        pltpu.SemaphoreType.DMA,
        pltpu.SemaphoreType.DMA,
        pltpu.SemaphoreType.DMA,
    ]


@functools.partial(
    pl.kernel,
    mesh=_mesh,
    out_type=jax.ShapeDtypeStruct((_F * _D * _B,), jnp.float32),
    scratch_types=_scratch,
    compiler_params=pltpu.CompilerParams(use_tc_tiling_on_sc=True),
)
def _emb_lookup(w1d_hbm, xt_hbm, out_hbm, spmem, asem, xbuf, xsem, *bufs):
    core = lax.axis_index("c")
    tid = lax.axis_index("s")
    sets = [bufs[7 * i: 7 * i + 7] for i in range(_UPF)]

    def field_of(j):
        return 2 * j + core

    def rs_of(f):
        r0 = (40000 * f) >> 7
        return jnp.minimum(r0, (_V // 128) - _NR)

    def fire_phase_a(j, slot):
        f = field_of(j)
        rs = rs_of(f)
        for dpl in range(2):
            src = dpl * _DPLANE + rs * 1024 + tid * _SHARE
            dstw = slot * _SSLOT + dpl * _SPLANE + tid * _SHARE
            pltpu.async_copy(w1d_hbm.at[pl.ds(src, _SHARE)],
                             spmem.at[pl.ds(dstw, _SHARE)], asem)

    def wait_phase_a():
        for _ in range(2):
            pltpu.make_async_copy(w1d_hbm.at[pl.ds(0, _SHARE)],
                                  spmem.at[pl.ds(0, _SHARE)], asem).wait()

    def fire_x(j, slot):
        f = field_of(j)
        pltpu.async_copy(xt_hbm.at[f, pl.ds(tid * 1024, 1024)],
                         xbuf.at[pl.ds(slot * 1024, 1024)], xsem)

    def wait_x():
        pltpu.make_async_copy(xt_hbm.at[0, pl.ds(0, 1024)],
                              xbuf.at[pl.ds(0, 1024)], xsem).wait()

    def stage(u, s, f, slot, addc, j):
        idx0, idx1, dst0, dst1, g0, g1, ws = s

        @pl.when(j > 0)
        def _():
            pltpu.make_async_copy(dst0, out_hbm.at[pl.ds(0, _HC)], ws).wait()
            pltpu.make_async_copy(dst1, out_hbm.at[pl.ds(0, _HC)], ws).wait()

        off = 40000 * f
        for g in range(_C // 16):
            r = xbuf[pl.ds(slot * 1024 + u * _C + 16 * g, 16)] + off
            base = ((r >> 7) << 10) + (r & 127) + addc
            sl = (g // 8) * 1024 + (g % 8) * 16
            for sub in range(8):
                idx0[pl.ds(sl + sub * 128, 16)] = base + (sub * 128)
                idx1[pl.ds(sl + sub * 128, 16)] = base + (_SPLANE + sub * 128)
        pltpu.async_copy(spmem.at[idx0], dst0, g0)
        pltpu.async_copy(spmem.at[idx1], dst1, g1)

    def drain(u, s, f):
        idx0, idx1, dst0, dst1, g0, g1, ws = s
        b0 = tid * 1024 + u * _C
        obase = f * (_D * _B) + (b0 // 128) * 1024
        pltpu.make_async_copy(spmem.at[idx0], dst0, g0).wait()
        pltpu.make_async_copy(spmem.at[idx1], dst1, g1).wait()
        pltpu.async_copy(dst0, out_hbm.at[pl.ds(obase, _HC)], ws)
        pltpu.async_copy(dst1, out_hbm.at[pl.ds(obase + 8 * _B, _HC)], ws)

    fire_phase_a(0, 0)
    fire_x(0, 0)

    def body(j, carry):
        slot = j & 1
        f = field_of(j)
        wait_phase_a()
        plsc.subcore_barrier()
        nxt = jnp.minimum(j + 1, _FPC - 1)
        fire_phase_a(nxt, 1 - slot)
        fire_x(nxt, 1 - slot)
        wait_x()
        addc = slot * _SSLOT - rs_of(f) * 1024
        for u in range(_UPF):
            stage(u, sets[u], f, slot, addc, j)
        for u in range(_UPF):
            drain(u, sets[u], f)
        return carry

    lax.fori_loop(0, _FPC, body, 0)
    wait_phase_a()
    wait_x()
    for s in sets:
        _, _, dst0, dst1, _, _, ws = s
        pltpu.make_async_copy(dst0, out_hbm.at[pl.ds(0, _HC)], ws).wait()
        pltpu.make_async_copy(dst1, out_hbm.at[pl.ds(0, _HC)], ws).wait()


def kernel(x, weight):
    wt = weight.T
    w1d = (wt.reshape(2, 8, _V // 128, 128)
             .transpose(0, 2, 1, 3)
             .reshape(-1))
    xt = x.T
    out1d = _emb_lookup(w1d, xt)
    return (out1d.reshape(_F, 2, _B // 128, 8, 128)
                 .transpose(2, 4, 0, 1, 3)
                 .reshape(_B, _F, _D))
